# baseline (speedup 1.0000x reference)
import functools

import jax
import jax.numpy as jnp
from jax import lax
from jax.experimental import pallas as pl
from jax.experimental.pallas import tpu as pltpu
from jax.experimental.pallas import tpu_sc as plsc

E = 8
K = 2
M = 1024
F = 4096
S = 2048
CAP = 512
SLOTS = E * CAP
XE_ROWS = SLOTS + 8
NW = 32
TPW = S // NW


def _route_body(x_ref, wg_ref, s0b_ref, s1b_ref, g0_ref, g1_ref, ll_ref):
    x = x_ref[...]
    wg = wg_ref[...]
    logits = jnp.dot(x, wg, preferred_element_type=jnp.float32)

    e_iota = lax.broadcasted_iota(jnp.int32, (S, E), 1).astype(jnp.float32)
    m1 = jnp.max(logits, axis=1, keepdims=True)
    a1 = jnp.min(jnp.where(logits == m1, e_iota, float(E)), axis=1,
                 keepdims=True)
    mask0 = e_iota == a1
    neg = jnp.where(mask0, -jnp.inf, logits)
    m2 = jnp.max(neg, axis=1, keepdims=True)
    a2 = jnp.min(jnp.where(neg == m2, e_iota, float(E)), axis=1,
                 keepdims=True)
    mask1 = e_iota == a2
    mask0f = mask0.astype(jnp.float32)
    mask1f = mask1.astype(jnp.float32)

    ex = jnp.exp(logits - m1)
    gates = ex / jnp.sum(ex, axis=1, keepdims=True)
    g0 = jnp.sum(gates * mask0f, axis=1, keepdims=True)
    g1 = jnp.sum(gates * mask1f, axis=1, keepdims=True)
    denom = jnp.maximum(g0 + g1, 1e-9)
    g0n = g0 / denom
    g1n = g1 / denom

    B = 256
    ri = lax.broadcasted_iota(jnp.int32, (B, B), 0)
    ci = lax.broadcasted_iota(jnp.int32, (B, B), 1)
    tri = (ri >= ci).astype(jnp.float32)

    def blocked_cumsum(m, acc0):
        acc = acc0
        outs = []
        for b in range(S // B):
            blk = m[b * B:(b + 1) * B, :]
            c = jnp.dot(tri, blk, preferred_element_type=jnp.float32) + acc
            outs.append(c)
            acc = c[B - 1:B, :]
        return jnp.concatenate(outs, axis=0), acc

    cum0, count0 = blocked_cumsum(mask0f, jnp.zeros((1, E), jnp.float32))
    cum1, _ = blocked_cumsum(mask1f, count0)

    loc0 = cum0 - 1.0
    loc1 = cum1 - 1.0
    loc0_s = jnp.sum(loc0 * mask0f, axis=1, keepdims=True)
    loc1_s = jnp.sum(loc1 * mask1f, axis=1, keepdims=True)
    kept0 = loc0_s < float(CAP)
    kept1 = loc1_s < float(CAP)
    slot0 = a1 * float(CAP) + loc0_s
    slot1 = a2 * float(CAP) + loc1_s

    s0b_ref[...] = jnp.where(kept0, slot0, float(SLOTS)).astype(jnp.int32)
    s1b_ref[...] = jnp.where(kept1, slot1, float(SLOTS)).astype(jnp.int32)
    g0_ref[...] = jnp.broadcast_to(jnp.where(kept0, g0n, 0.0), (S, 16))
    g1_ref[...] = jnp.broadcast_to(jnp.where(kept1, g1n, 0.0), (S, 16))

    me = jnp.sum(gates, axis=0, keepdims=True)
    ll_ref[0, 0] = jnp.sum(me * count0) * (E / (S * S))


def _route(x, wg, interpret=False):
    out_shapes = (
        jax.ShapeDtypeStruct((S, 1), jnp.int32),
        jax.ShapeDtypeStruct((S, 1), jnp.int32),
        jax.ShapeDtypeStruct((S, 16), jnp.float32),
        jax.ShapeDtypeStruct((S, 16), jnp.float32),
        jax.ShapeDtypeStruct((1, 1), jnp.float32),
    )
    out_specs = (
        pl.BlockSpec((S, 1), lambda: (0, 0)),
        pl.BlockSpec((S, 1), lambda: (0, 0)),
        pl.BlockSpec((S, 16), lambda: (0, 0)),
        pl.BlockSpec((S, 16), lambda: (0, 0)),
        pl.BlockSpec(memory_space=pltpu.SMEM),
    )
    return pl.pallas_call(
        _route_body,
        out_shape=out_shapes,
        out_specs=out_specs,
        interpret=interpret,
    )(x, wg)


DCH = 32
DNCH = TPW // DCH
HEXP = E // 2
HSLOTS = HEXP * CAP
XEH_ROWS = HSLOTS + 8


@functools.cache
def _get_dispatch(half):
    mesh = plsc.VectorSubcoreMesh(core_axis_name="c", subcore_axis_name="s")
    sbase = half * HSLOTS

    @functools.partial(
        pl.kernel,
        mesh=mesh,
        out_type=jax.ShapeDtypeStruct((XEH_ROWS, M), jnp.float32),
        scratch_types=(
            [pltpu.VMEM((DCH,), jnp.int32) for _ in range(2 * DNCH)]
            + [pltpu.VMEM((DCH, M), jnp.float32) for _ in range(DNCH)]
            + [pltpu.SemaphoreType.DMA for _ in range(3 * DNCH)]
        ),
    )
    def _dispatch(s0_hbm, s1_hbm, x_hbm, xe_hbm,
                  i00, i01, i10, i11, rows0, rows1,
                  sl0, sl1, ss00, ss01, ss10, ss11):
        idx0 = (i00, i01)
        idx1 = (i10, i11)
        rows = (rows0, rows1)
        sload = (sl0, sl1)
        sscat0 = (ss00, ss01)
        sscat1 = (ss10, ss11)
        wid = lax.axis_index("s") * 2 + lax.axis_index("c")
        tbase = wid * TPW
        loads = []
        for c in range(DNCH):
            base = tbase + c * DCH
            pltpu.sync_copy(s0_hbm.at[pl.ds(base, DCH)], idx0[c])
            pltpu.sync_copy(s1_hbm.at[pl.ds(base, DCH)], idx1[c])
            for buf in (idx0[c], idx1[c]):
                for j in range(DCH // 16):
                    v = buf[pl.ds(j * 16, 16)] - sbase
                    bad = (v < 0) | (v >= HSLOTS)
                    buf[pl.ds(j * 16, 16)] = jnp.where(bad, HSLOTS, v)
            loads.append(pltpu.async_copy(
                x_hbm.at[pl.ds(base, DCH)], rows[c], sload[c]))
        scats = []
        for c in range(DNCH):
            loads[c].wait()
            scats.append(pltpu.async_copy(
                rows[c], xe_hbm.at[idx0[c]], sscat0[c]))
            scats.append(pltpu.async_copy(
                rows[c], xe_hbm.at[idx1[c]], sscat1[c]))
        for cp in scats:
            cp.wait()

    return _dispatch


BF = 2048


def _ffn_body(xe_ref, w1_ref, w2_ref, out_ref):
    f = pl.program_id(1)
    xb = xe_ref[...].astype(jnp.bfloat16)
    w1b = w1_ref[0].astype(jnp.bfloat16)
    w2b = w2_ref[0].astype(jnp.bfloat16)
    h = jnp.maximum(
        jnp.dot(xb, w1b, preferred_element_type=jnp.float32), 0.0)
    contrib = jnp.dot(h.astype(jnp.bfloat16), w2b,
                      preferred_element_type=jnp.float32)

    @pl.when(f == 0)
    def _():
        out_ref[...] = contrib

    @pl.when(f != 0)
    def _():
        out_ref[...] += contrib


def _ffn_half(xe, w1, w2, half, eo_prev=None):
    ins = [xe, w1, w2]
    in_specs = [
        pl.BlockSpec((CAP, M), lambda e, f: (e, 0)),
        pl.BlockSpec((1, M, BF), lambda e, f, h=half: (e + HEXP * h, 0, f)),
        pl.BlockSpec((1, BF, M), lambda e, f, h=half: (e + HEXP * h, f, 0)),
    ]
    kwargs = {}
    if eo_prev is not None:
        ins.append(eo_prev)
        in_specs.append(pl.BlockSpec(memory_space=pl.ANY))
        kwargs["input_output_aliases"] = {3: 0}

    def body(xe_ref, w1_ref, w2_ref, *rest):
        out_ref = rest[-1]
        _ffn_body(xe_ref, w1_ref, w2_ref, out_ref)

    return pl.pallas_call(
        body,
        grid=(HEXP, F // BF),
        in_specs=in_specs,
        out_specs=pl.BlockSpec(
            (CAP, M), lambda e, f, h=half: (e + HEXP * h, 0)),
        out_shape=jax.ShapeDtypeStruct((SLOTS, M), jnp.float32),
        compiler_params=pltpu.CompilerParams(
            dimension_semantics=("parallel", "arbitrary")),
        **kwargs,
    )(*ins)


CH = 16
NCH = TPW // CH


@functools.cache
def _get_combine():
    mesh = plsc.VectorSubcoreMesh(core_axis_name="c", subcore_axis_name="s")

    @functools.partial(
        pl.kernel,
        mesh=mesh,
        out_type=jax.ShapeDtypeStruct((S, M), jnp.float32),
        scratch_types=(
            [pltpu.VMEM((CH,), jnp.int32) for _ in range(4)]
            + [pltpu.VMEM((CH, M), jnp.float32) for _ in range(4)]
            + [pltpu.VMEM((CH, M), jnp.float32) for _ in range(2)]
            + [pltpu.VMEM((TPW, 16), jnp.float32) for _ in range(2)]
            + [pltpu.SemaphoreType.DMA for _ in range(6)]
        ),
    )
    def _combine(s0_hbm, s1_hbm, g0_hbm, g1_hbm, eo_hbm, y_hbm,
                 i00, i01, i10, i11, r00, r01, r10, r11, y0, y1,
                 g0a, g1a, sg00, sg01, sg10, sg11, sy0, sy1):
        idx0 = (i00, i01)
        idx1 = (i10, i11)
        r0 = (r00, r01)
        r1 = (r10, r11)
        yb = (y0, y1)
        sg0 = (sg00, sg01)
        sg1 = (sg10, sg11)
        sy = (sy0, sy1)
        wid = lax.axis_index("s") * 2 + lax.axis_index("c")
        tbase = wid * TPW

        def start_chunk(c):
            sl = c & 1
            base = tbase + c * CH
            pltpu.sync_copy(s0_hbm.at[pl.ds(base, CH)], idx0[sl])
            pltpu.sync_copy(s1_hbm.at[pl.ds(base, CH)], idx1[sl])
            idx0[sl][...] = jnp.minimum(idx0[sl][...], SLOTS - 1)
            idx1[sl][...] = jnp.minimum(idx1[sl][...], SLOTS - 1)
            cp0 = pltpu.async_copy(eo_hbm.at[idx0[sl]], r0[sl], sg0[sl])
            cp1 = pltpu.async_copy(eo_hbm.at[idx1[sl]], r1[sl], sg1[sl])
            return cp0, cp1

        pend = start_chunk(0)
        pltpu.sync_copy(g0_hbm.at[pl.ds(tbase, TPW)], g0a)
        pltpu.sync_copy(g1_hbm.at[pl.ds(tbase, TPW)], g1a)
        ypend = [None, None]
        for c in range(NCH):
            sl = c & 1
            pend[0].wait()
            pend[1].wait()
            if c + 1 < NCH:
                pend = start_chunk(c + 1)
            if ypend[sl] is not None:
                ypend[sl].wait()

            def row_body(i, carry, c=c, sl=sl):
                ga = g0a[c * CH + i, :]
                gb = g1a[c * CH + i, :]
                for j in range(M // 16):
                    yb[sl][i, j * 16:(j + 1) * 16] = (
                        ga * r0[sl][i, j * 16:(j + 1) * 16]
                        + gb * r1[sl][i, j * 16:(j + 1) * 16])
                return carry

            lax.fori_loop(0, CH, row_body, 0)
            ypend[sl] = pltpu.async_copy(
                yb[sl], y_hbm.at[pl.ds(tbase + c * CH, CH)], sy[sl])
        ypend[0].wait()
        ypend[1].wait()

    return _combine


def kernel(x, wg, w1, w2):
    s0, s1, g0, g1, ll = _route(x, wg)
    s0 = s0.reshape(S)
    s1 = s1.reshape(S)
    xe0 = _get_dispatch(0)(s0, s1, x)
    xe1 = _get_dispatch(1)(s0, s1, x)
    eo = _ffn_half(xe0, w1, w2, 0)
    eo = _ffn_half(xe1, w1, w2, 1, eo)
    y = _get_combine()(s0, s1, g0, g1, eo)
    return y, ll[0, 0]


# speedup vs baseline: 2.0923x; 2.0923x over previous
import functools

import jax
import jax.numpy as jnp
from jax import lax
from jax.experimental import pallas as pl
from jax.experimental.pallas import tpu as pltpu
from jax.experimental.pallas import tpu_sc as plsc

E = 8
K = 2
M = 1024
F = 4096
S = 2048
CAP = 512
SLOTS = E * CAP
XE_ROWS = SLOTS + 8
NW = 32
TPW = S // NW


def _route_body(x_ref, wg_ref, s0b_ref, s1b_ref, g0_ref, g1_ref, ll_ref):
    x = x_ref[...]
    wg = wg_ref[...]
    logits = jnp.dot(x, wg, preferred_element_type=jnp.float32)

    e_iota = lax.broadcasted_iota(jnp.int32, (S, E), 1).astype(jnp.float32)
    m1 = jnp.max(logits, axis=1, keepdims=True)
    a1 = jnp.min(jnp.where(logits == m1, e_iota, float(E)), axis=1,
                 keepdims=True)
    mask0 = e_iota == a1
    neg = jnp.where(mask0, -jnp.inf, logits)
    m2 = jnp.max(neg, axis=1, keepdims=True)
    a2 = jnp.min(jnp.where(neg == m2, e_iota, float(E)), axis=1,
                 keepdims=True)
    mask1 = e_iota == a2
    mask0f = mask0.astype(jnp.float32)
    mask1f = mask1.astype(jnp.float32)

    ex = jnp.exp(logits - m1)
    gates = ex / jnp.sum(ex, axis=1, keepdims=True)
    g0 = jnp.sum(gates * mask0f, axis=1, keepdims=True)
    g1 = jnp.sum(gates * mask1f, axis=1, keepdims=True)
    denom = jnp.maximum(g0 + g1, 1e-9)
    g0n = g0 / denom
    g1n = g1 / denom

    B = 256
    ri = lax.broadcasted_iota(jnp.int32, (B, B), 0)
    ci = lax.broadcasted_iota(jnp.int32, (B, B), 1)
    tri = (ri >= ci).astype(jnp.float32)

    def blocked_cumsum(m, acc0):
        acc = acc0
        outs = []
        for b in range(S // B):
            blk = m[b * B:(b + 1) * B, :]
            c = jnp.dot(tri, blk, preferred_element_type=jnp.float32) + acc
            outs.append(c)
            acc = c[B - 1:B, :]
        return jnp.concatenate(outs, axis=0), acc

    cum0, count0 = blocked_cumsum(mask0f, jnp.zeros((1, E), jnp.float32))
    cum1, _ = blocked_cumsum(mask1f, count0)

    loc0 = cum0 - 1.0
    loc1 = cum1 - 1.0
    loc0_s = jnp.sum(loc0 * mask0f, axis=1, keepdims=True)
    loc1_s = jnp.sum(loc1 * mask1f, axis=1, keepdims=True)
    kept0 = loc0_s < float(CAP)
    kept1 = loc1_s < float(CAP)
    slot0 = a1 * float(CAP) + loc0_s
    slot1 = a2 * float(CAP) + loc1_s

    s0b_ref[...] = jnp.where(kept0, slot0, float(SLOTS)).astype(jnp.int32)
    s1b_ref[...] = jnp.where(kept1, slot1, float(SLOTS)).astype(jnp.int32)
    g0_ref[...] = jnp.broadcast_to(jnp.where(kept0, g0n, 0.0), (S, 16))
    g1_ref[...] = jnp.broadcast_to(jnp.where(kept1, g1n, 0.0), (S, 16))

    me = jnp.sum(gates, axis=0, keepdims=True)
    ll_ref[...] = jnp.sum(me * count0, keepdims=True).reshape(1, 1) * (
        E / (S * S))


def _route(x, wg, interpret=False):
    out_shapes = (
        jax.ShapeDtypeStruct((S, 1), jnp.int32),
        jax.ShapeDtypeStruct((S, 1), jnp.int32),
        jax.ShapeDtypeStruct((S, 16), jnp.float32),
        jax.ShapeDtypeStruct((S, 16), jnp.float32),
        jax.ShapeDtypeStruct((1, 1), jnp.float32),
    )
    out_specs = (
        pl.BlockSpec((S, 1), lambda: (0, 0)),
        pl.BlockSpec((S, 1), lambda: (0, 0)),
        pl.BlockSpec((S, 16), lambda: (0, 0)),
        pl.BlockSpec((S, 16), lambda: (0, 0)),
        pl.BlockSpec((1, 1), lambda: (0, 0)),
    )
    return pl.pallas_call(
        _route_body,
        out_shape=out_shapes,
        out_specs=out_specs,
        interpret=interpret,
    )(x, wg)


DCH = 32
DNCH = TPW // DCH


@functools.cache
def _get_dispatch():
    mesh = plsc.VectorSubcoreMesh(core_axis_name="c", subcore_axis_name="s")

    @functools.partial(
        pl.kernel,
        mesh=mesh,
        out_type=jax.ShapeDtypeStruct((XE_ROWS, M), jnp.float32),
        scratch_types=(
            [pltpu.VMEM((DCH,), jnp.int32) for _ in range(2 * DNCH)]
            + [pltpu.VMEM((DCH, M), jnp.float32) for _ in range(DNCH)]
            + [pltpu.SemaphoreType.DMA for _ in range(3 * DNCH)]
        ),
    )
    def _dispatch(s0_hbm, s1_hbm, x_hbm, xe_hbm,
                  i00, i01, i10, i11, rows0, rows1,
                  sl0, sl1, ss00, ss01, ss10, ss11):
        idx0 = (i00, i01)
        idx1 = (i10, i11)
        rows = (rows0, rows1)
        sload = (sl0, sl1)
        sscat0 = (ss00, ss01)
        sscat1 = (ss10, ss11)
        wid = lax.axis_index("s") * 2 + lax.axis_index("c")
        tbase = wid * TPW
        loads = []
        for c in range(DNCH):
            base = tbase + c * DCH
            pltpu.sync_copy(s0_hbm.at[pl.ds(base, DCH)], idx0[c])
            pltpu.sync_copy(s1_hbm.at[pl.ds(base, DCH)], idx1[c])
            loads.append(pltpu.async_copy(
                x_hbm.at[pl.ds(base, DCH)], rows[c], sload[c]))
        scats = []
        for c in range(DNCH):
            loads[c].wait()
            scats.append(pltpu.async_copy(
                rows[c], xe_hbm.at[idx0[c]], sscat0[c]))
            scats.append(pltpu.async_copy(
                rows[c], xe_hbm.at[idx1[c]], sscat1[c]))
        for cp in scats:
            cp.wait()

    return _dispatch


BF = 2048


def _ffn_body(xe_ref, w1_ref, w2_ref, out_ref):
    f = pl.program_id(1)
    xb = xe_ref[...].astype(jnp.bfloat16)
    w1b = w1_ref[0].astype(jnp.bfloat16)
    w2b = w2_ref[0].astype(jnp.bfloat16)
    h = jnp.maximum(
        jnp.dot(xb, w1b, preferred_element_type=jnp.float32), 0.0)
    contrib = jnp.dot(h.astype(jnp.bfloat16), w2b,
                      preferred_element_type=jnp.float32)

    @pl.when(f == 0)
    def _():
        out_ref[...] = contrib

    @pl.when(f != 0)
    def _():
        out_ref[...] += contrib


def _ffn(xe, w1, w2):
    return pl.pallas_call(
        _ffn_body,
        grid=(E, F // BF),
        in_specs=[
            pl.BlockSpec((CAP, M), lambda e, f: (e, 0)),
            pl.BlockSpec((1, M, BF), lambda e, f: (e, 0, f)),
            pl.BlockSpec((1, BF, M), lambda e, f: (e, f, 0)),
        ],
        out_specs=pl.BlockSpec((CAP, M), lambda e, f: (e, 0)),
        out_shape=jax.ShapeDtypeStruct((SLOTS, M), jnp.float32),
        compiler_params=pltpu.CompilerParams(
            dimension_semantics=("parallel", "arbitrary")),
    )(xe, w1, w2)


CH = 16
NCH = TPW // CH


@functools.cache
def _get_combine():
    mesh = plsc.VectorSubcoreMesh(core_axis_name="c", subcore_axis_name="s")

    @functools.partial(
        pl.kernel,
        mesh=mesh,
        out_type=jax.ShapeDtypeStruct((S, M), jnp.float32),
        scratch_types=(
            [pltpu.VMEM((CH,), jnp.int32) for _ in range(4)]
            + [pltpu.VMEM((CH, M), jnp.float32) for _ in range(4)]
            + [pltpu.VMEM((CH, M), jnp.float32) for _ in range(2)]
            + [pltpu.VMEM((TPW, 16), jnp.float32) for _ in range(2)]
            + [pltpu.SemaphoreType.DMA for _ in range(6)]
        ),
    )
    def _combine(s0_hbm, s1_hbm, g0_hbm, g1_hbm, eo_hbm, y_hbm,
                 i00, i01, i10, i11, r00, r01, r10, r11, y0, y1,
                 g0a, g1a, sg00, sg01, sg10, sg11, sy0, sy1):
        idx0 = (i00, i01)
        idx1 = (i10, i11)
        r0 = (r00, r01)
        r1 = (r10, r11)
        yb = (y0, y1)
        sg0 = (sg00, sg01)
        sg1 = (sg10, sg11)
        sy = (sy0, sy1)
        wid = lax.axis_index("s") * 2 + lax.axis_index("c")
        tbase = wid * TPW

        def start_chunk(c):
            sl = c & 1
            base = tbase + c * CH
            pltpu.sync_copy(s0_hbm.at[pl.ds(base, CH)], idx0[sl])
            pltpu.sync_copy(s1_hbm.at[pl.ds(base, CH)], idx1[sl])
            idx0[sl][...] = jnp.minimum(idx0[sl][...], SLOTS - 1)
            idx1[sl][...] = jnp.minimum(idx1[sl][...], SLOTS - 1)
            cp0 = pltpu.async_copy(eo_hbm.at[idx0[sl]], r0[sl], sg0[sl])
            cp1 = pltpu.async_copy(eo_hbm.at[idx1[sl]], r1[sl], sg1[sl])
            return cp0, cp1

        pend = start_chunk(0)
        pltpu.sync_copy(g0_hbm.at[pl.ds(tbase, TPW)], g0a)
        pltpu.sync_copy(g1_hbm.at[pl.ds(tbase, TPW)], g1a)
        ypend = [None, None]
        for c in range(NCH):
            sl = c & 1
            pend[0].wait()
            pend[1].wait()
            if c + 1 < NCH:
                pend = start_chunk(c + 1)
            if ypend[sl] is not None:
                ypend[sl].wait()

            def row_body(i, carry, c=c, sl=sl):
                ga = g0a[c * CH + i, :]
                gb = g1a[c * CH + i, :]
                for j in range(M // 16):
                    yb[sl][i, j * 16:(j + 1) * 16] = (
                        ga * r0[sl][i, j * 16:(j + 1) * 16]
                        + gb * r1[sl][i, j * 16:(j + 1) * 16])
                return carry

            lax.fori_loop(0, CH, row_body, 0)
            ypend[sl] = pltpu.async_copy(
                yb[sl], y_hbm.at[pl.ds(tbase + c * CH, CH)], sy[sl])
        ypend[0].wait()
        ypend[1].wait()

    return _combine


def kernel(x, wg, w1, w2):
    s0, s1, g0, g1, ll = _route(x, wg)
    s0 = s0.reshape(S)
    s1 = s1.reshape(S)
    xe = _get_dispatch()(s0, s1, x)
    eo = _ffn(xe, w1, w2)
    y = _get_combine()(s0, s1, g0, g1, eo)
    return y, ll[0, 0]
